# baseline (speedup 1.0000x reference)
import functools

import jax
import jax.numpy as jnp
from jax import lax
from jax.experimental import pallas as pl
from jax.experimental.pallas import tpu as pltpu
from jax.experimental.pallas import tpu_sc as plsc

N = 10000
D = 128
G = 64
NC = 2
NS = 16
NW = NC * NS
EB = 128
ROWS_PER_SUB = 640
ACC_ROWS = NS * ROWS_PER_SUB
ROW_BLK = 1000
SPLIT0_NUM = 28


BD = 8
BE = 2


def _sc_deg(dstp3, bpw):
    mesh = plsc.VectorSubcoreMesh(core_axis_name="c", subcore_axis_name="s")

    @functools.partial(
        pl.kernel,
        out_type=jax.ShapeDtypeStruct((NC * N,), jnp.float32),
        mesh=mesh,
        scratch_types=[
            pltpu.VMEM((BD, 1, EB), jnp.int32),
            pltpu.VMEM((EB,), jnp.float32),
            pltpu.VMEM((ROWS_PER_SUB,), jnp.float32),
            pltpu.VMEM_SHARED((ACC_ROWS,), jnp.float32),
        ],
    )
    def deg_kernel(dstp_hbm, out_hbm, dbuf, ones_buf, zbuf, dacc):
        c = lax.axis_index("c")
        s = lax.axis_index("s")
        w = c * NS + s

        def fill_ones(i, carry):
            ones_buf[pl.ds(i * 16, 16)] = jnp.full((16,), 1.0, jnp.float32)
            return carry

        lax.fori_loop(0, EB // 16, fill_ones, 0)

        def fill_zero(i, carry):
            zbuf[pl.ds(i * 16, 16)] = jnp.zeros((16,), jnp.float32)
            return carry

        lax.fori_loop(0, ROWS_PER_SUB // 16, fill_zero, 0)
        pltpu.sync_copy(zbuf, dacc.at[pl.ds(s * ROWS_PER_SUB, ROWS_PER_SUB)])
        plsc.subcore_barrier()

        def body(i, carry):
            row0 = w * bpw + i * BD
            pltpu.sync_copy(dstp_hbm.at[pl.ds(row0, BD)], dbuf)
            for k in range(BD):
                pltpu.sync_copy(ones_buf, dacc.at[dbuf.at[k, 0]], add=True)
            return carry

        lax.fori_loop(0, bpw // BD, body, 0)
        plsc.subcore_barrier()

        r0 = s * ROWS_PER_SUB
        n_last = N - (NS - 1) * ROWS_PER_SUB
        pltpu.sync_copy(dacc.at[pl.ds(r0, ROWS_PER_SUB)], zbuf)

        @pl.when(s < NS - 1)
        def _():
            pltpu.sync_copy(zbuf, out_hbm.at[pl.ds(c * N + r0, ROWS_PER_SUB)])

        @pl.when(s == NS - 1)
        def _():
            pltpu.sync_copy(zbuf.at[pl.ds(0, n_last)],
                            out_hbm.at[pl.ds(c * N + r0, n_last)])

    return deg_kernel(dstp3)


def _sc_edge(z, srcp, dstp3, bps0, bps1):
    mesh = plsc.VectorSubcoreMesh(core_axis_name="c", subcore_axis_name="s")

    @functools.partial(
        pl.kernel,
        out_type=jax.ShapeDtypeStruct((NC, N, D), jnp.float32),
        mesh=mesh,
        scratch_types=[
            pltpu.VMEM((BE * EB,), jnp.int32),
            pltpu.VMEM((BE, 1, EB), jnp.int32),
            pltpu.VMEM((BE, EB, D), jnp.float32),
            pltpu.VMEM_SHARED((ACC_ROWS, D), jnp.float32),
            pltpu.SemaphoreType.DMA,
        ],
    )
    def edge_kernel(z_hbm, srcp_hbm, dstp_hbm, out_hbm, sbuf, dbuf, gbuf,
                    acc, sem):
        c = lax.axis_index("c")
        s = lax.axis_index("s")
        J = jnp.where(c == 0, bps0, bps1)
        blk_w = jnp.where(c == 0, s * bps0, NS * bps0 + s * bps1)

        def zero_row(i, carry):
            for j in range(D // 16):
                gbuf[0, i, pl.ds(j * 16, 16)] = jnp.zeros((16,), jnp.float32)
            return carry

        lax.fori_loop(0, EB, zero_row, 0)
        for k in range(ROWS_PER_SUB // EB):
            pltpu.sync_copy(gbuf.at[0],
                            acc.at[pl.ds(s * ROWS_PER_SUB + k * EB, EB)])
        plsc.subcore_barrier()

        def body(i, carry):
            row0 = blk_w + i * BE
            pltpu.sync_copy(srcp_hbm.at[pl.ds(row0 * EB, BE * EB)], sbuf)
            pltpu.sync_copy(dstp_hbm.at[pl.ds(row0, BE)], dbuf)
            for k in range(BE):
                pltpu.async_copy(z_hbm.at[sbuf.at[pl.ds(k * EB, EB)]],
                                 gbuf.at[k], sem).wait()
                pltpu.sync_copy(gbuf.at[k], acc.at[dbuf.at[k, 0]], add=True)
            return carry

        lax.fori_loop(0, J // BE, body, 0)
        plsc.subcore_barrier()

        r0 = s * ROWS_PER_SUB
        n_last = N - (NS - 1) * ROWS_PER_SUB

        @pl.when(s < NS - 1)
        def _():
            pltpu.sync_copy(acc.at[pl.ds(r0, ROWS_PER_SUB)],
                            out_hbm.at[c, pl.ds(r0, ROWS_PER_SUB)])

        @pl.when(s == NS - 1)
        def _():
            pltpu.sync_copy(acc.at[pl.ds(r0, n_last)],
                            out_hbm.at[c, pl.ds(r0, n_last)])

    return edge_kernel(z, srcp, dstp3)


def _tc_scale_mm(x, W1, dinv):
    def body(x_ref, w_ref, dinv_ref, o_ref):
        o_ref[...] = dinv_ref[...] * jnp.dot(
            x_ref[...], w_ref[...], preferred_element_type=jnp.float32)

    return pl.pallas_call(
        body,
        grid=(N // ROW_BLK,),
        in_specs=[
            pl.BlockSpec((ROW_BLK, D), lambda i: (i, 0)),
            pl.BlockSpec((D, D), lambda i: (0, 0)),
            pl.BlockSpec((ROW_BLK, 1), lambda i: (i, 0)),
        ],
        out_specs=pl.BlockSpec((ROW_BLK, D), lambda i: (i, 0)),
        out_shape=jax.ShapeDtypeStruct((N, D), jnp.float32),
    )(x, W1, dinv)


def _tc_combine(aggP, z1, dinv, b1, W2):
    def body(agg_ref, z_ref, dinv_ref, b_ref, w_ref, o_ref):
        agg = agg_ref[0] + agg_ref[1] + z_ref[...]
        h = jnp.maximum(dinv_ref[...] * agg + b_ref[...], 0.0)
        o_ref[...] = dinv_ref[...] * jnp.dot(
            h, w_ref[...], preferred_element_type=jnp.float32)

    return pl.pallas_call(
        body,
        grid=(N // ROW_BLK,),
        in_specs=[
            pl.BlockSpec((NC, ROW_BLK, D), lambda i: (0, i, 0)),
            pl.BlockSpec((ROW_BLK, D), lambda i: (i, 0)),
            pl.BlockSpec((ROW_BLK, 1), lambda i: (i, 0)),
            pl.BlockSpec((1, D), lambda i: (0, 0)),
            pl.BlockSpec((D, D), lambda i: (0, 0)),
        ],
        out_specs=pl.BlockSpec((ROW_BLK, D), lambda i: (i, 0)),
        out_shape=jax.ShapeDtypeStruct((N, D), jnp.float32),
    )(aggP, z1, dinv, b1, W2)


def _tc_final(aggP, z2, dinv, b2, batchf, Wfc, bfc, gamma, beta):
    def body(agg_ref, z_ref, dinv_ref, b_ref, bt_ref, wfc_ref, bfc_ref,
             g_ref, be_ref, o_ref, psum, cntm):
        i = pl.program_id(0)

        @pl.when(i == 0)
        def _():
            psum[...] = jnp.zeros_like(psum)
            cntm[...] = jnp.zeros_like(cntm)

        agg = agg_ref[0] + agg_ref[1] + z_ref[...]
        h = jnp.maximum(dinv_ref[...] * agg + b_ref[...], 0.0)
        iota = lax.broadcasted_iota(jnp.int32, (ROW_BLK, 128), 1)
        onehot = (bt_ref[...] == iota).astype(jnp.float32)
        dn = (((0,), (0,)), ((), ()))
        psum[...] += lax.dot_general(onehot, h, dn,
                                     preferred_element_type=jnp.float32)
        cntm[...] += lax.dot_general(onehot, jnp.ones_like(h), dn,
                                     preferred_element_type=jnp.float32)

        @pl.when(i == pl.num_programs(0) - 1)
        def _():
            p = psum[0:G, :] / jnp.maximum(cntm[0:G, :], 1.0)
            fc = jnp.dot(p, wfc_ref[...],
                         preferred_element_type=jnp.float32) + bfc_ref[...]
            mu = jnp.mean(fc, axis=1, keepdims=True)
            var = jnp.mean((fc - mu) ** 2, axis=1, keepdims=True)
            o_ref[...] = (fc - mu) * lax.rsqrt(var + 1e-5) * g_ref[...] + be_ref[...]

    return pl.pallas_call(
        body,
        grid=(N // ROW_BLK,),
        in_specs=[
            pl.BlockSpec((NC, ROW_BLK, D), lambda i: (0, i, 0)),
            pl.BlockSpec((ROW_BLK, D), lambda i: (i, 0)),
            pl.BlockSpec((ROW_BLK, 1), lambda i: (i, 0)),
            pl.BlockSpec((1, D), lambda i: (0, 0)),
            pl.BlockSpec((ROW_BLK, 1), lambda i: (i, 0)),
            pl.BlockSpec((D, D), lambda i: (0, 0)),
            pl.BlockSpec((1, D), lambda i: (0, 0)),
            pl.BlockSpec((1, D), lambda i: (0, 0)),
            pl.BlockSpec((1, D), lambda i: (0, 0)),
        ],
        out_specs=pl.BlockSpec((G, D), lambda i: (0, 0)),
        out_shape=jax.ShapeDtypeStruct((G, D), jnp.float32),
        scratch_shapes=[
            pltpu.VMEM((128, D), jnp.float32),
            pltpu.VMEM((128, D), jnp.float32),
        ],
    )(aggP, z2, dinv, b2, batchf, Wfc, bfc, gamma, beta)


def kernel(x, edge_index, batch, W1, b1, W2, b2, Wfc, bfc, gamma, beta):
    E = edge_index.shape[1]
    bpw = BD * pl.cdiv(E, NW * EB * BD)
    pad = bpw * EB * NW - E
    bps = 2 * bpw
    bps0 = (bps * SPLIT0_NUM // 40) // BE * BE
    bps1 = bps - bps0

    src = edge_index[0]
    dst = edge_index[1]
    srcp = jnp.concatenate([src, jnp.zeros((pad,), jnp.int32)])
    dstp = jnp.concatenate([dst, jnp.full((pad,), N, jnp.int32)])
    dstp3 = dstp.reshape(-1, 1, EB)

    degP = _sc_deg(dstp3, bpw)
    deg = degP[:N] + degP[N:] + 1.0
    dinv = lax.rsqrt(deg)[:, None]

    z1 = _tc_scale_mm(x, W1, dinv)
    agg1 = _sc_edge(z1, srcp, dstp3, bps0, bps1)
    z2 = _tc_combine(agg1, z1, dinv, b1.reshape(1, D), W2)
    agg2 = _sc_edge(z2, srcp, dstp3, bps0, bps1)
    batchf = batch[:, None]
    return _tc_final(agg2, z2, dinv, b2.reshape(1, D), batchf,
                     Wfc, bfc.reshape(1, D), gamma.reshape(1, D),
                     beta.reshape(1, D))


# speedup vs baseline: 1.3307x; 1.3307x over previous
import functools

import jax
import jax.numpy as jnp
from jax import lax
from jax.experimental import pallas as pl
from jax.experimental.pallas import tpu as pltpu
from jax.experimental.pallas import tpu_sc as plsc

N = 10000
D = 128
G = 64
NC = 2
NS = 16
NW = NC * NS
EB = 128
ROWS_PER_SUB = 640
ACC_ROWS = NS * ROWS_PER_SUB
ROW_BLK = 1000
SPLIT0_NUM = 28


def _sc_deg(dstp, bpw, epw):
    mesh = plsc.VectorSubcoreMesh(core_axis_name="c", subcore_axis_name="s")

    @functools.partial(
        pl.kernel,
        out_type=jax.ShapeDtypeStruct((NC * N,), jnp.float32),
        mesh=mesh,
        scratch_types=[
            pltpu.VMEM((EB,), jnp.int32),
            pltpu.VMEM((EB,), jnp.float32),
            pltpu.VMEM((ROWS_PER_SUB,), jnp.float32),
            pltpu.VMEM_SHARED((ACC_ROWS,), jnp.float32),
        ],
    )
    def deg_kernel(dstp_hbm, out_hbm, dbuf, ones_buf, zbuf, dacc):
        c = lax.axis_index("c")
        s = lax.axis_index("s")
        w = c * NS + s

        def fill_ones(i, carry):
            ones_buf[pl.ds(i * 16, 16)] = jnp.full((16,), 1.0, jnp.float32)
            return carry

        lax.fori_loop(0, EB // 16, fill_ones, 0)

        def fill_zero(i, carry):
            zbuf[pl.ds(i * 16, 16)] = jnp.zeros((16,), jnp.float32)
            return carry

        lax.fori_loop(0, ROWS_PER_SUB // 16, fill_zero, 0)
        pltpu.sync_copy(zbuf, dacc.at[pl.ds(s * ROWS_PER_SUB, ROWS_PER_SUB)])
        plsc.subcore_barrier()

        def body(i, carry):
            base = w * epw + i * EB
            pltpu.sync_copy(dstp_hbm.at[pl.ds(base, EB)], dbuf)
            pltpu.sync_copy(ones_buf, dacc.at[dbuf], add=True)
            return carry

        lax.fori_loop(0, bpw, body, 0)
        plsc.subcore_barrier()

        r0 = s * ROWS_PER_SUB
        n_last = N - (NS - 1) * ROWS_PER_SUB
        pltpu.sync_copy(dacc.at[pl.ds(r0, ROWS_PER_SUB)], zbuf)

        @pl.when(s < NS - 1)
        def _():
            pltpu.sync_copy(zbuf, out_hbm.at[pl.ds(c * N + r0, ROWS_PER_SUB)])

        @pl.when(s == NS - 1)
        def _():
            pltpu.sync_copy(zbuf.at[pl.ds(0, n_last)],
                            out_hbm.at[pl.ds(c * N + r0, n_last)])

    return deg_kernel(dstp)


def _sc_edge(z, srcp, dstp, bps0, bps1):
    mesh = plsc.VectorSubcoreMesh(core_axis_name="c", subcore_axis_name="s")

    @functools.partial(
        pl.kernel,
        out_type=jax.ShapeDtypeStruct((NC, N, D), jnp.float32),
        mesh=mesh,
        scratch_types=[
            pltpu.VMEM((EB,), jnp.int32),
            pltpu.VMEM((EB,), jnp.int32),
            pltpu.VMEM((EB, D), jnp.float32),
            pltpu.VMEM_SHARED((ACC_ROWS, D), jnp.float32),
            pltpu.SemaphoreType.DMA,
        ],
    )
    def edge_kernel(z_hbm, srcp_hbm, dstp_hbm, out_hbm, sbuf, dbuf, gbuf,
                    acc, sem):
        c = lax.axis_index("c")
        s = lax.axis_index("s")
        J = jnp.where(c == 0, bps0, bps1)
        base_w = jnp.where(c == 0, s * bps0, NS * bps0 + s * bps1) * EB

        def zero_row(i, carry):
            for j in range(D // 16):
                gbuf[i, pl.ds(j * 16, 16)] = jnp.zeros((16,), jnp.float32)
            return carry

        lax.fori_loop(0, EB, zero_row, 0)
        for k in range(ROWS_PER_SUB // EB):
            pltpu.sync_copy(gbuf, acc.at[pl.ds(s * ROWS_PER_SUB + k * EB, EB)])
        plsc.subcore_barrier()

        def body(i, carry):
            base = base_w + i * EB
            pltpu.sync_copy(srcp_hbm.at[pl.ds(base, EB)], sbuf)
            pltpu.sync_copy(dstp_hbm.at[pl.ds(base, EB)], dbuf)
            pltpu.async_copy(z_hbm.at[sbuf], gbuf, sem).wait()
            pltpu.sync_copy(gbuf, acc.at[dbuf], add=True)
            return carry

        lax.fori_loop(0, J, body, 0)
        plsc.subcore_barrier()

        r0 = s * ROWS_PER_SUB
        n_last = N - (NS - 1) * ROWS_PER_SUB

        @pl.when(s < NS - 1)
        def _():
            pltpu.sync_copy(acc.at[pl.ds(r0, ROWS_PER_SUB)],
                            out_hbm.at[c, pl.ds(r0, ROWS_PER_SUB)])

        @pl.when(s == NS - 1)
        def _():
            pltpu.sync_copy(acc.at[pl.ds(r0, n_last)],
                            out_hbm.at[c, pl.ds(r0, n_last)])

    return edge_kernel(z, srcp, dstp)


def _tc_scale_mm(x, W1, dinv):
    def body(x_ref, w_ref, dinv_ref, o_ref):
        o_ref[...] = dinv_ref[...] * jnp.dot(
            x_ref[...], w_ref[...], preferred_element_type=jnp.float32)

    return pl.pallas_call(
        body,
        grid=(N // ROW_BLK,),
        in_specs=[
            pl.BlockSpec((ROW_BLK, D), lambda i: (i, 0)),
            pl.BlockSpec((D, D), lambda i: (0, 0)),
            pl.BlockSpec((ROW_BLK, 1), lambda i: (i, 0)),
        ],
        out_specs=pl.BlockSpec((ROW_BLK, D), lambda i: (i, 0)),
        out_shape=jax.ShapeDtypeStruct((N, D), jnp.float32),
    )(x, W1, dinv)


def _tc_combine(aggP, z1, dinv, b1, W2):
    def body(agg_ref, z_ref, dinv_ref, b_ref, w_ref, o_ref):
        agg = agg_ref[0] + agg_ref[1] + z_ref[...]
        h = jnp.maximum(dinv_ref[...] * agg + b_ref[...], 0.0)
        o_ref[...] = dinv_ref[...] * jnp.dot(
            h, w_ref[...], preferred_element_type=jnp.float32)

    return pl.pallas_call(
        body,
        grid=(N // ROW_BLK,),
        in_specs=[
            pl.BlockSpec((NC, ROW_BLK, D), lambda i: (0, i, 0)),
            pl.BlockSpec((ROW_BLK, D), lambda i: (i, 0)),
            pl.BlockSpec((ROW_BLK, 1), lambda i: (i, 0)),
            pl.BlockSpec((1, D), lambda i: (0, 0)),
            pl.BlockSpec((D, D), lambda i: (0, 0)),
        ],
        out_specs=pl.BlockSpec((ROW_BLK, D), lambda i: (i, 0)),
        out_shape=jax.ShapeDtypeStruct((N, D), jnp.float32),
    )(aggP, z1, dinv, b1, W2)


def _tc_final(aggP, z2, dinv, b2, batchf, Wfc, bfc, gamma, beta):
    def body(agg_ref, z_ref, dinv_ref, b_ref, bt_ref, wfc_ref, bfc_ref,
             g_ref, be_ref, o_ref, psum, cntm):
        i = pl.program_id(0)

        @pl.when(i == 0)
        def _():
            psum[...] = jnp.zeros_like(psum)
            cntm[...] = jnp.zeros_like(cntm)

        agg = agg_ref[0] + agg_ref[1] + z_ref[...]
        h = jnp.maximum(dinv_ref[...] * agg + b_ref[...], 0.0)
        iota = lax.broadcasted_iota(jnp.int32, (ROW_BLK, 128), 1)
        onehot = (bt_ref[...] == iota).astype(jnp.float32)
        dn = (((0,), (0,)), ((), ()))
        psum[...] += lax.dot_general(onehot, h, dn,
                                     preferred_element_type=jnp.float32)
        cntm[...] += lax.dot_general(onehot, jnp.ones_like(h), dn,
                                     preferred_element_type=jnp.float32)

        @pl.when(i == pl.num_programs(0) - 1)
        def _():
            p = psum[0:G, :] / jnp.maximum(cntm[0:G, :], 1.0)
            fc = jnp.dot(p, wfc_ref[...],
                         preferred_element_type=jnp.float32) + bfc_ref[...]
            mu = jnp.mean(fc, axis=1, keepdims=True)
            var = jnp.mean((fc - mu) ** 2, axis=1, keepdims=True)
            o_ref[...] = (fc - mu) * lax.rsqrt(var + 1e-5) * g_ref[...] + be_ref[...]

    return pl.pallas_call(
        body,
        grid=(N // ROW_BLK,),
        in_specs=[
            pl.BlockSpec((NC, ROW_BLK, D), lambda i: (0, i, 0)),
            pl.BlockSpec((ROW_BLK, D), lambda i: (i, 0)),
            pl.BlockSpec((ROW_BLK, 1), lambda i: (i, 0)),
            pl.BlockSpec((1, D), lambda i: (0, 0)),
            pl.BlockSpec((ROW_BLK, 1), lambda i: (i, 0)),
            pl.BlockSpec((D, D), lambda i: (0, 0)),
            pl.BlockSpec((1, D), lambda i: (0, 0)),
            pl.BlockSpec((1, D), lambda i: (0, 0)),
            pl.BlockSpec((1, D), lambda i: (0, 0)),
        ],
        out_specs=pl.BlockSpec((G, D), lambda i: (0, 0)),
        out_shape=jax.ShapeDtypeStruct((G, D), jnp.float32),
        scratch_shapes=[
            pltpu.VMEM((128, D), jnp.float32),
            pltpu.VMEM((128, D), jnp.float32),
        ],
    )(aggP, z2, dinv, b2, batchf, Wfc, bfc, gamma, beta)


def kernel(x, edge_index, batch, W1, b1, W2, b2, Wfc, bfc, gamma, beta):
    E = edge_index.shape[1]
    bpw = pl.cdiv(E, NW * EB)
    epw = bpw * EB
    pad = epw * NW - E
    bps = 2 * bpw
    bps0 = bps * SPLIT0_NUM // 40
    bps1 = bps - bps0

    src = edge_index[0]
    dst = edge_index[1]
    srcp = jnp.concatenate([src, jnp.zeros((pad,), jnp.int32)])
    dstp = jnp.concatenate([dst, jnp.full((pad,), N, jnp.int32)])

    degP = _sc_deg(dstp, bpw, epw)
    deg = degP[:N] + degP[N:] + 1.0
    dinv = lax.rsqrt(deg)[:, None]

    z1 = _tc_scale_mm(x, W1, dinv)
    agg1 = _sc_edge(z1, srcp, dstp, bps0, bps1)
    z2 = _tc_combine(agg1, z1, dinv, b1.reshape(1, D), W2)
    agg2 = _sc_edge(z2, srcp, dstp, bps0, bps1)
    batchf = batch[:, None]
    return _tc_final(agg2, z2, dinv, b2.reshape(1, D), batchf,
                     Wfc, bfc.reshape(1, D), gamma.reshape(1, D),
                     beta.reshape(1, D))
